# Initial kernel scaffold; baseline (speedup 1.0000x reference)
import jax
import jax.numpy as jnp
from jax.experimental import pallas as pl


def kernel(x):
    raise NotImplementedError("write your pallas kernel here")


# speedup vs baseline: 51.8115x; 51.8115x over previous
import numpy as np
import jax
import jax.numpy as jnp
from jax import lax
from jax.experimental import pallas as pl
from jax.experimental.pallas import tpu as pltpu
from jax.experimental.pallas import tpu_sc as plsc

R = 6
H, W = 64, 128
Hh, Ww = 32, 64
NCH = R * 5
NIMG = 8 * 32
NTASK = NIMG * NCH
NWORK = 32
TPW = NTASK // NWORK
NOUT = Hh * Ww
CHW = H * W
IMW = NCH * CHW
TAB = 7 * NOUT
BUF = 8448

_RUNA = 8192
_LEFT = 8256
_CORNER = 8320
_COLB = 8384


def _gather_table() -> np.ndarray:
    hg, wg = np.meshgrid(np.arange(Hh), np.arange(Ww), indexing="ij")
    dh = np.array([0, 1, 1, 0, -1, -1, 0])
    dw = np.array([0, 0, 1, 1, 0, -1, -1])
    nh = 1 + 2 * hg[..., None] + dh
    nw = 1 + 2 * wg[..., None] + dw
    off = np.empty((Hh, Ww, 7), np.int64)
    m = (nh >= 1) & (nw >= 1)
    off[m] = (nh[m] - 1) * W + (nw[m] - 1)
    m = (nw == 0) & (nh >= 1)
    off[m] = _CORNER - nh[m]
    m = (nh == 0) & (nw == 0)
    off[m] = _CORNER
    m = (nh == 0) & (nw >= 1) & (nw <= 64)
    off[m] = _RUNA + nw[m] - 1
    m = (nh == 0) & (nw >= 65)
    off[m] = (_COLB + 128) - nw[m]
    tab = off.reshape(NOUT, 7).T.astype(np.int32)
    colb = (np.arange(64, dtype=np.int32) * W + (W - 1))
    misc = np.empty((32,), np.int32)
    misc[0:16] = 127
    misc[0:4] = (127, 8064, _COLB, _LEFT)
    misc[16:32] = 63
    misc[16:18] = (63, 1984)
    return np.concatenate([tab.ravel(), colb, misc])


_IDX_TAB = _gather_table()
_COLB_OFF = TAB
_MISC_OFF = TAB + 64


def _body(x_hbm, idx_hbm, out_hbm, idxv, buf, cidx, ob, sem):
    wid = lax.axis_index("s") * 2 + lax.axis_index("c")
    pltpu.sync_copy(idx_hbm, idxv)
    zeros = jnp.zeros((16,), jnp.float32)
    sc_idx = idxv[pl.ds(_MISC_OFF, 16)]
    oc_idx = idxv[pl.ds(_MISC_OFF + 16, 16)]

    def task(k, carry):
        t = wid * TPW + k
        n = t // NCH
        ch = t - n * NCH
        r = ch // 5
        f = ch - r * 5
        fm1 = (f + 4) % 5
        base = n * NCH
        c_a = base + r * 5 + fm1
        c_b = base + ((r + 1) % R) * 5 + fm1
        c_l = base + ((r + 5) % R) * 5 + fm1
        for q in range(4):
            cidx[pl.ds(q * 16, 16)] = (
                idxv[pl.ds(_COLB_OFF + q * 16, 16)] + c_b * CHW
            )
        pltpu.sync_copy(x_hbm.at[pl.ds(t * CHW, CHW)], buf.at[pl.ds(0, CHW)])
        pltpu.sync_copy(
            x_hbm.at[pl.ds(c_a * CHW + (H - 1) * W + Ww, Ww)],
            buf.at[pl.ds(_RUNA, Ww)],
        )
        pltpu.sync_copy(
            x_hbm.at[pl.ds(c_l * CHW + (H - 1) * W, Ww)],
            buf.at[pl.ds(_LEFT, Ww)],
        )
        pltpu.sync_copy(
            x_hbm.at[pl.ds(n * IMW, 8)], buf.at[pl.ds(_CORNER, 8)]
        )
        pltpu.async_copy(x_hbm.at[cidx], buf.at[pl.ds(_COLB, 64)], sem).wait()
        plsc.store_scatter(buf, [sc_idx], zeros)

        def vb(v, c):
            g = []
            for kk in range(7):
                fi = idxv[pl.ds(kk * NOUT + v * 16, 16)]
                g.append(plsc.load_gather(buf, [fi]))
            acc = ((g[0] + g[1]) + (g[2] + g[3])) + ((g[4] + g[5]) + g[6])
            ob[pl.ds(v * 16, 16)] = acc * np.float32(1.0 / 7.0)
            return c

        lax.fori_loop(0, NOUT // 16, vb, 0)
        plsc.store_scatter(ob, [oc_idx], zeros)
        pltpu.sync_copy(ob, out_hbm.at[pl.ds(t * NOUT, NOUT)])
        return carry

    lax.fori_loop(0, TPW, task, 0)


@jax.jit
def kernel(x):
    x1 = x.reshape(NTASK * CHW)
    idx = jnp.asarray(_IDX_TAB)
    run = pl.kernel(
        _body,
        out_type=jax.ShapeDtypeStruct((NTASK * NOUT,), jnp.float32),
        mesh=plsc.VectorSubcoreMesh(core_axis_name="c", subcore_axis_name="s"),
        compiler_params=pltpu.CompilerParams(
            use_tc_tiling_on_sc=False, needs_layout_passes=False
        ),
        scratch_types=[
            pltpu.VMEM((TAB + 96,), jnp.int32),
            pltpu.VMEM((BUF,), jnp.float32),
            pltpu.VMEM((64,), jnp.int32),
            pltpu.VMEM((NOUT,), jnp.float32),
            pltpu.SemaphoreType.DMA,
        ],
    )
    out = run(x1, idx)
    return out.reshape(8, 32, R, 5, Hh, Ww)


# speedup vs baseline: 120.1240x; 2.3185x over previous
import numpy as np
import jax
import jax.numpy as jnp
from jax import lax
from jax.experimental import pallas as pl
from jax.experimental.pallas import tpu as pltpu
from jax.experimental.pallas import tpu_sc as plsc

R = 6
H, W = 64, 128
Hh, Ww = 32, 64
NCH = R * 5
NIMG = 8 * 32
NTASK = NIMG * NCH
NWORK = 32
TPW = NTASK // NWORK
NOUT = Hh * Ww
CHW = H * W
IMW = NCH * CHW
TAB = 7 * NOUT
BUF = 8448

_RUNA = 8192
_LEFT = 8256
_CORNER = 8320
_COLB = 8384


def _gather_table() -> np.ndarray:
    hg, wg = np.meshgrid(np.arange(Hh), np.arange(Ww), indexing="ij")
    dh = np.array([0, 1, 1, 0, -1, -1, 0])
    dw = np.array([0, 0, 1, 1, 0, -1, -1])
    nh = 1 + 2 * hg[..., None] + dh
    nw = 1 + 2 * wg[..., None] + dw
    off = np.empty((Hh, Ww, 7), np.int64)
    m = (nh >= 1) & (nw >= 1)
    off[m] = (nh[m] - 1) * W + (nw[m] - 1)
    m = (nw == 0) & (nh >= 1)
    off[m] = _CORNER - nh[m]
    m = (nh == 0) & (nw == 0)
    off[m] = _CORNER
    m = (nh == 0) & (nw >= 1) & (nw <= 64)
    off[m] = _RUNA + nw[m] - 1
    m = (nh == 0) & (nw >= 65)
    off[m] = (_COLB + 128) - nw[m]
    tab = off.reshape(NOUT, 7).T.astype(np.int32)
    colb = (np.arange(64, dtype=np.int32) * W + (W - 1))
    misc = np.empty((32,), np.int32)
    misc[0:16] = 127
    misc[0:4] = (127, 8064, _COLB, _LEFT)
    misc[16:32] = 63
    misc[16:18] = (63, 1984)
    return np.concatenate([tab.ravel(), colb, misc])


_IDX_TAB = _gather_table()
_COLB_OFF = TAB
_MISC_OFF = TAB + 64


def _body(x_hbm, idx_hbm, out_hbm, idxv, buf0, buf1, cidx0, cidx1, ob, sem0, sem1):
    wid = lax.axis_index("s") * 2 + lax.axis_index("c")
    pltpu.sync_copy(idx_hbm, idxv)
    zeros = jnp.zeros((16,), jnp.float32)
    sc_idx = idxv[pl.ds(_MISC_OFF, 16)]
    oc_idx = idxv[pl.ds(_MISC_OFF + 16, 16)]
    tbase = wid * TPW

    def neighbors(t):
        n = t // NCH
        ch = t - n * NCH
        r = ch // 5
        f = ch - r * 5
        fm1 = (f + 4) % 5
        base = n * NCH
        c_a = base + r * 5 + fm1
        c_b = base + ((r + 1) % R) * 5 + fm1
        c_l = base + ((r + 5) % R) * 5 + fm1
        return n, c_a, c_b, c_l

    def fetch_descs(t, buf, cidx, sem):
        n, c_a, c_b, c_l = neighbors(t)
        return [
            pltpu.make_async_copy(
                x_hbm.at[pl.ds(t * CHW, CHW)], buf.at[pl.ds(0, CHW)], sem
            ),
            pltpu.make_async_copy(
                x_hbm.at[pl.ds(c_a * CHW + (H - 1) * W + Ww, Ww)],
                buf.at[pl.ds(_RUNA, Ww)],
                sem,
            ),
            pltpu.make_async_copy(
                x_hbm.at[pl.ds(c_l * CHW + (H - 1) * W, Ww)],
                buf.at[pl.ds(_LEFT, Ww)],
                sem,
            ),
            pltpu.make_async_copy(
                x_hbm.at[pl.ds(n * IMW, 8)], buf.at[pl.ds(_CORNER, 8)], sem
            ),
            pltpu.make_async_copy(x_hbm.at[cidx], buf.at[pl.ds(_COLB, 64)], sem),
        ]

    def issue(t, buf, cidx, sem):
        _, _, c_b, _ = neighbors(t)
        for q in range(4):
            cidx[pl.ds(q * 16, 16)] = (
                idxv[pl.ds(_COLB_OFF + q * 16, 16)] + c_b * CHW
            )
        for d in fetch_descs(t, buf, cidx, sem):
            d.start()

    def compute(t, buf):
        plsc.store_scatter(buf, [sc_idx], zeros)

        def vb(v, c):
            g = []
            for kk in range(7):
                fi = idxv[pl.ds(kk * NOUT + v * 16, 16)]
                g.append(plsc.load_gather(buf, [fi]))
            acc = ((g[0] + g[1]) + (g[2] + g[3])) + ((g[4] + g[5]) + g[6])
            ob[pl.ds(v * 16, 16)] = acc * np.float32(1.0 / 7.0)
            return c

        lax.fori_loop(0, NOUT // 16, vb, 0)
        plsc.store_scatter(ob, [oc_idx], zeros)
        pltpu.sync_copy(ob, out_hbm.at[pl.ds(t * NOUT, NOUT)])

    issue(tbase, buf0, cidx0, sem0)

    def pair(k, carry):
        t0 = tbase + 2 * k
        t1 = t0 + 1
        issue(t1, buf1, cidx1, sem1)
        for d in fetch_descs(t0, buf0, cidx0, sem0):
            d.wait()
        compute(t0, buf0)

        @pl.when(k < TPW // 2 - 1)
        def _():
            issue(t0 + 2, buf0, cidx0, sem0)

        for d in fetch_descs(t1, buf1, cidx1, sem1):
            d.wait()
        compute(t1, buf1)
        return carry

    lax.fori_loop(0, TPW // 2, pair, 0)


@jax.jit
def kernel(x):
    x1 = x.reshape(NTASK * CHW)
    idx = jnp.asarray(_IDX_TAB)
    run = pl.kernel(
        _body,
        out_type=jax.ShapeDtypeStruct((NTASK * NOUT,), jnp.float32),
        mesh=plsc.VectorSubcoreMesh(core_axis_name="c", subcore_axis_name="s"),
        compiler_params=pltpu.CompilerParams(
            use_tc_tiling_on_sc=False, needs_layout_passes=False
        ),
        scratch_types=[
            pltpu.VMEM((TAB + 96,), jnp.int32),
            pltpu.VMEM((BUF,), jnp.float32),
            pltpu.VMEM((BUF,), jnp.float32),
            pltpu.VMEM((64,), jnp.int32),
            pltpu.VMEM((64,), jnp.int32),
            pltpu.VMEM((NOUT,), jnp.float32),
            pltpu.SemaphoreType.DMA,
            pltpu.SemaphoreType.DMA,
        ],
    )
    out = run(x1, idx)
    return out.reshape(8, 32, R, 5, Hh, Ww)


# speedup vs baseline: 148.8089x; 1.2388x over previous
import numpy as np
import jax
import jax.numpy as jnp
from jax import lax
from jax.experimental import pallas as pl
from jax.experimental.pallas import tpu as pltpu
from jax.experimental.pallas import tpu_sc as plsc

R = 6
H, W = 64, 128
Hh, Ww = 32, 64
NCH = R * 5
NIMG = 8 * 32
NTASK = NIMG * NCH
NWORK = 32
TPW = NTASK // NWORK
NOUT = Hh * Ww
CHW = H * W
IMW = NCH * CHW
TAB = 7 * NOUT
BUF = 8448

_RUNA = 8192
_LEFT = 8256
_CORNER = 8320
_COLB = 8384


def _gather_table() -> np.ndarray:
    hg, wg = np.meshgrid(np.arange(Hh), np.arange(Ww), indexing="ij")
    dh = np.array([0, 1, 1, 0, -1, -1, 0])
    dw = np.array([0, 0, 1, 1, 0, -1, -1])
    nh = 1 + 2 * hg[..., None] + dh
    nw = 1 + 2 * wg[..., None] + dw
    off = np.empty((Hh, Ww, 7), np.int64)
    m = (nh >= 1) & (nw >= 1)
    off[m] = (nh[m] - 1) * W + (nw[m] - 1)
    m = (nw == 0) & (nh >= 1)
    off[m] = _CORNER - nh[m]
    m = (nh == 0) & (nw == 0)
    off[m] = _CORNER
    m = (nh == 0) & (nw >= 1) & (nw <= 64)
    off[m] = _RUNA + nw[m] - 1
    m = (nh == 0) & (nw >= 65)
    off[m] = (_COLB + 128) - nw[m]
    tab = off.reshape(NOUT, 7).T.astype(np.int32)
    colb = (np.arange(64, dtype=np.int32) * W + (W - 1))
    misc = np.empty((32,), np.int32)
    misc[0:16] = 127
    misc[0:4] = (127, 8064, _COLB, _LEFT)
    misc[16:32] = 63
    misc[16:18] = (63, 1984)
    return np.concatenate([tab.ravel(), colb, misc])


_IDX_TAB = _gather_table()
_COLB_OFF = TAB
_MISC_OFF = TAB + 64


def _body(x_hbm, idx_hbm, out_hbm, idxv, buf0, buf1, cidx0, cidx1, ob, sem0, sem1):
    wid = lax.axis_index("s") * 2 + lax.axis_index("c")
    pltpu.sync_copy(idx_hbm, idxv)
    zeros = jnp.zeros((16,), jnp.float32)
    sc_idx = idxv[pl.ds(_MISC_OFF, 16)]
    oc_idx = idxv[pl.ds(_MISC_OFF + 16, 16)]
    tbase = wid * TPW

    def neighbors(t):
        n = t // NCH
        ch = t - n * NCH
        r = ch // 5
        f = ch - r * 5
        fm1 = (f + 4) % 5
        base = n * NCH
        c_a = base + r * 5 + fm1
        c_b = base + ((r + 1) % R) * 5 + fm1
        c_l = base + ((r + 5) % R) * 5 + fm1
        return n, c_a, c_b, c_l

    def fetch_descs(t, buf, cidx, sem):
        n, c_a, c_b, c_l = neighbors(t)
        return [
            pltpu.make_async_copy(
                x_hbm.at[pl.ds(t * CHW, CHW)], buf.at[pl.ds(0, CHW)], sem
            ),
            pltpu.make_async_copy(
                x_hbm.at[pl.ds(c_a * CHW + (H - 1) * W + Ww, Ww)],
                buf.at[pl.ds(_RUNA, Ww)],
                sem,
            ),
            pltpu.make_async_copy(
                x_hbm.at[pl.ds(c_l * CHW + (H - 1) * W, Ww)],
                buf.at[pl.ds(_LEFT, Ww)],
                sem,
            ),
            pltpu.make_async_copy(
                x_hbm.at[pl.ds(n * IMW, 8)], buf.at[pl.ds(_CORNER, 8)], sem
            ),
            pltpu.make_async_copy(x_hbm.at[cidx], buf.at[pl.ds(_COLB, 64)], sem),
        ]

    def issue(t, buf, cidx, sem):
        _, _, c_b, _ = neighbors(t)
        for q in range(4):
            cidx[pl.ds(q * 16, 16)] = (
                idxv[pl.ds(_COLB_OFF + q * 16, 16)] + c_b * CHW
            )
        for d in fetch_descs(t, buf, cidx, sem):
            d.start()

    def compute(t, buf):
        plsc.store_scatter(buf, [sc_idx], zeros)

        @plsc.parallel_loop(0, NOUT // 16, unroll=4)
        def vb(v):
            g = []
            for kk in range(7):
                fi = idxv[pl.ds(kk * NOUT + v * 16, 16)]
                g.append(plsc.load_gather(buf, [fi]))
            acc = ((g[0] + g[1]) + (g[2] + g[3])) + ((g[4] + g[5]) + g[6])
            ob[pl.ds(v * 16, 16)] = acc * np.float32(1.0 / 7.0)
        plsc.store_scatter(ob, [oc_idx], zeros)
        pltpu.sync_copy(ob, out_hbm.at[pl.ds(t * NOUT, NOUT)])

    issue(tbase, buf0, cidx0, sem0)

    def pair(k, carry):
        t0 = tbase + 2 * k
        t1 = t0 + 1
        issue(t1, buf1, cidx1, sem1)
        for d in fetch_descs(t0, buf0, cidx0, sem0):
            d.wait()
        compute(t0, buf0)

        @pl.when(k < TPW // 2 - 1)
        def _():
            issue(t0 + 2, buf0, cidx0, sem0)

        for d in fetch_descs(t1, buf1, cidx1, sem1):
            d.wait()
        compute(t1, buf1)
        return carry

    lax.fori_loop(0, TPW // 2, pair, 0)


@jax.jit
def kernel(x):
    x1 = x.reshape(NTASK * CHW)
    idx = jnp.asarray(_IDX_TAB)
    run = pl.kernel(
        _body,
        out_type=jax.ShapeDtypeStruct((NTASK * NOUT,), jnp.float32),
        mesh=plsc.VectorSubcoreMesh(core_axis_name="c", subcore_axis_name="s"),
        compiler_params=pltpu.CompilerParams(
            use_tc_tiling_on_sc=False, needs_layout_passes=False
        ),
        scratch_types=[
            pltpu.VMEM((TAB + 96,), jnp.int32),
            pltpu.VMEM((BUF,), jnp.float32),
            pltpu.VMEM((BUF,), jnp.float32),
            pltpu.VMEM((64,), jnp.int32),
            pltpu.VMEM((64,), jnp.int32),
            pltpu.VMEM((NOUT,), jnp.float32),
            pltpu.SemaphoreType.DMA,
            pltpu.SemaphoreType.DMA,
        ],
    )
    out = run(x1, idx)
    return out.reshape(8, 32, R, 5, Hh, Ww)


# speedup vs baseline: 149.3151x; 1.0034x over previous
import numpy as np
import jax
import jax.numpy as jnp
from jax import lax
from jax.experimental import pallas as pl
from jax.experimental.pallas import tpu as pltpu
from jax.experimental.pallas import tpu_sc as plsc

R = 6
H, W = 64, 128
Hh, Ww = 32, 64
NCH = R * 5
NIMG = 8 * 32
NTASK = NIMG * NCH
NWORK = 32
TPW = NTASK // NWORK
NOUT = Hh * Ww
CHW = H * W
IMW = NCH * CHW
TAB = 7 * NOUT
BUF = 8448

_RUNA = 8192
_LEFT = 8256
_CORNER = 8320
_COLB = 8384


def _gather_table() -> np.ndarray:
    hg, wg = np.meshgrid(np.arange(Hh), np.arange(Ww), indexing="ij")
    dh = np.array([0, 1, 1, 0, -1, -1, 0])
    dw = np.array([0, 0, 1, 1, 0, -1, -1])
    nh = 1 + 2 * hg[..., None] + dh
    nw = 1 + 2 * wg[..., None] + dw
    off = np.empty((Hh, Ww, 7), np.int64)
    m = (nh >= 1) & (nw >= 1)
    off[m] = (nh[m] - 1) * W + (nw[m] - 1)
    m = (nw == 0) & (nh >= 1)
    off[m] = _CORNER - nh[m]
    m = (nh == 0) & (nw == 0)
    off[m] = _CORNER
    m = (nh == 0) & (nw >= 1) & (nw <= 64)
    off[m] = _RUNA + nw[m] - 1
    m = (nh == 0) & (nw >= 65)
    off[m] = (_COLB + 128) - nw[m]
    tab = off.reshape(NOUT, 7).T.astype(np.int32)
    colb = (np.arange(64, dtype=np.int32) * W + (W - 1))
    misc = np.empty((32,), np.int32)
    misc[0:16] = 127
    misc[0:4] = (127, 8064, _COLB, _LEFT)
    misc[16:32] = 63
    misc[16:18] = (63, 1984)
    return np.concatenate([tab.ravel(), colb, misc])


_IDX_TAB = _gather_table()
_COLB_OFF = TAB
_MISC_OFF = TAB + 64


def _body(x_hbm, idx_hbm, out_hbm, idxv, buf0, buf1, cidx0, cidx1, ob, sem0, sem1):
    wid = lax.axis_index("s") * 2 + lax.axis_index("c")
    pltpu.sync_copy(idx_hbm, idxv)
    zeros = jnp.zeros((16,), jnp.float32)
    sc_idx = idxv[pl.ds(_MISC_OFF, 16)]
    oc_idx = idxv[pl.ds(_MISC_OFF + 16, 16)]
    tbase = wid * TPW

    def neighbors(t):
        n = t // NCH
        ch = t - n * NCH
        r = ch // 5
        f = ch - r * 5
        fm1 = (f + 4) % 5
        base = n * NCH
        c_a = base + r * 5 + fm1
        c_b = base + ((r + 1) % R) * 5 + fm1
        c_l = base + ((r + 5) % R) * 5 + fm1
        return n, c_a, c_b, c_l

    def fetch_descs(t, buf, cidx, sem):
        n, c_a, c_b, c_l = neighbors(t)
        return [
            pltpu.make_async_copy(
                x_hbm.at[pl.ds(t * CHW, CHW)], buf.at[pl.ds(0, CHW)], sem
            ),
            pltpu.make_async_copy(
                x_hbm.at[pl.ds(c_a * CHW + (H - 1) * W + Ww, Ww)],
                buf.at[pl.ds(_RUNA, Ww)],
                sem,
            ),
            pltpu.make_async_copy(
                x_hbm.at[pl.ds(c_l * CHW + (H - 1) * W, Ww)],
                buf.at[pl.ds(_LEFT, Ww)],
                sem,
            ),
            pltpu.make_async_copy(
                x_hbm.at[pl.ds(n * IMW, 8)], buf.at[pl.ds(_CORNER, 8)], sem
            ),
            pltpu.make_async_copy(x_hbm.at[cidx], buf.at[pl.ds(_COLB, 64)], sem),
        ]

    def issue(t, buf, cidx, sem):
        _, _, c_b, _ = neighbors(t)
        for q in range(4):
            cidx[pl.ds(q * 16, 16)] = (
                idxv[pl.ds(_COLB_OFF + q * 16, 16)] + c_b * CHW
            )
        for d in fetch_descs(t, buf, cidx, sem):
            d.start()

    def compute(t, buf):
        plsc.store_scatter(buf, [sc_idx], zeros)

        @plsc.parallel_loop(0, NOUT // 16, unroll=8)
        def vb(v):
            g = []
            for kk in range(7):
                fi = idxv[pl.ds(kk * NOUT + v * 16, 16)]
                g.append(plsc.load_gather(buf, [fi]))
            acc = ((g[0] + g[1]) + (g[2] + g[3])) + ((g[4] + g[5]) + g[6])
            ob[pl.ds(v * 16, 16)] = acc * np.float32(1.0 / 7.0)
        plsc.store_scatter(ob, [oc_idx], zeros)
        pltpu.sync_copy(ob, out_hbm.at[pl.ds(t * NOUT, NOUT)])

    issue(tbase, buf0, cidx0, sem0)

    def pair(k, carry):
        t0 = tbase + 2 * k
        t1 = t0 + 1
        issue(t1, buf1, cidx1, sem1)
        for d in fetch_descs(t0, buf0, cidx0, sem0):
            d.wait()
        compute(t0, buf0)

        @pl.when(k < TPW // 2 - 1)
        def _():
            issue(t0 + 2, buf0, cidx0, sem0)

        for d in fetch_descs(t1, buf1, cidx1, sem1):
            d.wait()
        compute(t1, buf1)
        return carry

    lax.fori_loop(0, TPW // 2, pair, 0)


@jax.jit
def kernel(x):
    x1 = x.reshape(NTASK * CHW)
    idx = jnp.asarray(_IDX_TAB)
    run = pl.kernel(
        _body,
        out_type=jax.ShapeDtypeStruct((NTASK * NOUT,), jnp.float32),
        mesh=plsc.VectorSubcoreMesh(core_axis_name="c", subcore_axis_name="s"),
        compiler_params=pltpu.CompilerParams(
            use_tc_tiling_on_sc=False, needs_layout_passes=False
        ),
        scratch_types=[
            pltpu.VMEM((TAB + 96,), jnp.int32),
            pltpu.VMEM((BUF,), jnp.float32),
            pltpu.VMEM((BUF,), jnp.float32),
            pltpu.VMEM((64,), jnp.int32),
            pltpu.VMEM((64,), jnp.int32),
            pltpu.VMEM((NOUT,), jnp.float32),
            pltpu.SemaphoreType.DMA,
            pltpu.SemaphoreType.DMA,
        ],
    )
    out = run(x1, idx)
    return out.reshape(8, 32, R, 5, Hh, Ww)


# speedup vs baseline: 158.2670x; 1.0600x over previous
import numpy as np
import jax
import jax.numpy as jnp
from jax import lax
from jax.experimental import pallas as pl
from jax.experimental.pallas import tpu as pltpu
from jax.experimental.pallas import tpu_sc as plsc

R = 6
H, W = 64, 128
Hh, Ww = 32, 64
NCH = R * 5
NIMG = 8 * 32
NTASK = NIMG * NCH
NWORK = 32
TPW = NTASK // NWORK
NOUT = Hh * Ww
CHW = H * W
IMW = NCH * CHW
TAB = 7 * NOUT
BUF = 8448

_RUNA = 8192
_LEFT = 8256
_CORNER = 8320
_COLB = 8384


def _gather_table() -> np.ndarray:
    hg, wg = np.meshgrid(np.arange(Hh), np.arange(Ww), indexing="ij")
    dh = np.array([0, 1, 1, 0, -1, -1, 0])
    dw = np.array([0, 0, 1, 1, 0, -1, -1])
    nh = 1 + 2 * hg[..., None] + dh
    nw = 1 + 2 * wg[..., None] + dw
    off = np.empty((Hh, Ww, 7), np.int64)
    m = (nh >= 1) & (nw >= 1)
    off[m] = (nh[m] - 1) * W + (nw[m] - 1)
    m = (nw == 0) & (nh >= 1)
    off[m] = _CORNER - nh[m]
    m = (nh == 0) & (nw == 0)
    off[m] = _CORNER
    m = (nh == 0) & (nw >= 1) & (nw <= 64)
    off[m] = _RUNA + nw[m] - 1
    m = (nh == 0) & (nw >= 65)
    off[m] = (_COLB + 128) - nw[m]
    tab = off.reshape(NOUT, 7).T.astype(np.int32)
    colb = (np.arange(64, dtype=np.int32) * W + (W - 1))
    misc = np.empty((32,), np.int32)
    misc[0:16] = 127
    misc[0:4] = (127, 8064, _COLB, _LEFT)
    misc[16:32] = 63
    misc[16:18] = (63, 1984)
    return np.concatenate([tab.ravel(), colb, misc])


_IDX_TAB = _gather_table()
_COLB_OFF = TAB
_MISC_OFF = TAB + 64


def _body(
    x_hbm, idx_hbm, out_hbm, idxv, buf0, buf1, cidx0, cidx1, ob0, ob1,
    sem0, sem1, osem0, osem1,
):
    wid = lax.axis_index("s") * 2 + lax.axis_index("c")
    pltpu.sync_copy(idx_hbm, idxv)
    zeros = jnp.zeros((16,), jnp.float32)
    sc_idx = idxv[pl.ds(_MISC_OFF, 16)]
    oc_idx = idxv[pl.ds(_MISC_OFF + 16, 16)]
    tbase = wid * TPW

    def neighbors(t):
        n = t // NCH
        ch = t - n * NCH
        r = ch // 5
        f = ch - r * 5
        fm1 = (f + 4) % 5
        base = n * NCH
        c_a = base + r * 5 + fm1
        c_b = base + ((r + 1) % R) * 5 + fm1
        c_l = base + ((r + 5) % R) * 5 + fm1
        return n, c_a, c_b, c_l

    def fetch_descs(t, buf, cidx, sem):
        n, c_a, c_b, c_l = neighbors(t)
        return [
            pltpu.make_async_copy(
                x_hbm.at[pl.ds(t * CHW, CHW)], buf.at[pl.ds(0, CHW)], sem
            ),
            pltpu.make_async_copy(
                x_hbm.at[pl.ds(c_a * CHW + (H - 1) * W + Ww, Ww)],
                buf.at[pl.ds(_RUNA, Ww)],
                sem,
            ),
            pltpu.make_async_copy(
                x_hbm.at[pl.ds(c_l * CHW + (H - 1) * W, Ww)],
                buf.at[pl.ds(_LEFT, Ww)],
                sem,
            ),
            pltpu.make_async_copy(
                x_hbm.at[pl.ds(n * IMW, 8)], buf.at[pl.ds(_CORNER, 8)], sem
            ),
            pltpu.make_async_copy(x_hbm.at[cidx], buf.at[pl.ds(_COLB, 64)], sem),
        ]

    def issue(t, buf, cidx, sem):
        _, _, c_b, _ = neighbors(t)
        for q in range(4):
            cidx[pl.ds(q * 16, 16)] = (
                idxv[pl.ds(_COLB_OFF + q * 16, 16)] + c_b * CHW
            )
        for d in fetch_descs(t, buf, cidx, sem):
            d.start()

    def out_desc(t, ob, osem):
        return pltpu.make_async_copy(ob, out_hbm.at[pl.ds(t * NOUT, NOUT)], osem)

    def compute(k, t, buf, ob, osem):
        plsc.store_scatter(buf, [sc_idx], zeros)

        @pl.when(k > 0)
        def _():
            out_desc(t - 2, ob, osem).wait()

        @plsc.parallel_loop(0, NOUT // 16, unroll=8)
        def vb(v):
            g = []
            for kk in range(7):
                fi = idxv[pl.ds(kk * NOUT + v * 16, 16)]
                g.append(plsc.load_gather(buf, [fi]))
            acc = ((g[0] + g[1]) + (g[2] + g[3])) + ((g[4] + g[5]) + g[6])
            ob[pl.ds(v * 16, 16)] = acc * np.float32(1.0 / 7.0)
        plsc.store_scatter(ob, [oc_idx], zeros)
        out_desc(t, ob, osem).start()

    issue(tbase, buf0, cidx0, sem0)

    def pair(k, carry):
        t0 = tbase + 2 * k
        t1 = t0 + 1
        issue(t1, buf1, cidx1, sem1)
        for d in fetch_descs(t0, buf0, cidx0, sem0):
            d.wait()
        compute(k, t0, buf0, ob0, osem0)

        @pl.when(k < TPW // 2 - 1)
        def _():
            issue(t0 + 2, buf0, cidx0, sem0)

        for d in fetch_descs(t1, buf1, cidx1, sem1):
            d.wait()
        compute(k, t1, buf1, ob1, osem1)
        return carry

    lax.fori_loop(0, TPW // 2, pair, 0)
    out_desc(tbase + TPW - 2, ob0, osem0).wait()
    out_desc(tbase + TPW - 1, ob1, osem1).wait()


@jax.jit
def kernel(x):
    x1 = x.reshape(NTASK * CHW)
    idx = jnp.asarray(_IDX_TAB)
    run = pl.kernel(
        _body,
        out_type=jax.ShapeDtypeStruct((NTASK * NOUT,), jnp.float32),
        mesh=plsc.VectorSubcoreMesh(core_axis_name="c", subcore_axis_name="s"),
        compiler_params=pltpu.CompilerParams(
            use_tc_tiling_on_sc=False, needs_layout_passes=False
        ),
        scratch_types=[
            pltpu.VMEM((TAB + 96,), jnp.int32),
            pltpu.VMEM((BUF,), jnp.float32),
            pltpu.VMEM((BUF,), jnp.float32),
            pltpu.VMEM((64,), jnp.int32),
            pltpu.VMEM((64,), jnp.int32),
            pltpu.VMEM((NOUT,), jnp.float32),
            pltpu.VMEM((NOUT,), jnp.float32),
            pltpu.SemaphoreType.DMA,
            pltpu.SemaphoreType.DMA,
            pltpu.SemaphoreType.DMA,
            pltpu.SemaphoreType.DMA,
        ],
    )
    out = run(x1, idx)
    return out.reshape(8, 32, R, 5, Hh, Ww)


# speedup vs baseline: 168.4974x; 1.0646x over previous
import numpy as np
import jax
import jax.numpy as jnp
from jax import lax
from jax.experimental import pallas as pl
from jax.experimental.pallas import tpu as pltpu
from jax.experimental.pallas import tpu_sc as plsc

R = 6
H, W = 64, 128
Hh, Ww = 32, 64
NCH = R * 5
NIMG = 8 * 32
NTASK = NIMG * NCH
NWORK = 32
TPW = NTASK // NWORK
NOUT = Hh * Ww
CHW = H * W
IMW = NCH * CHW
TAB = 7 * NOUT
BUF = 8448

_RUNA = 8192
_LEFT = 8256
_CORNER = 8320
_COLB = 8384


def _gather_table() -> np.ndarray:
    hg, wg = np.meshgrid(np.arange(Hh), np.arange(Ww), indexing="ij")
    dh = np.array([0, 1, 1, 0, -1, -1, 0])
    dw = np.array([0, 0, 1, 1, 0, -1, -1])
    nh = 1 + 2 * hg[..., None] + dh
    nw = 1 + 2 * wg[..., None] + dw
    off = np.empty((Hh, Ww, 7), np.int64)
    m = (nh >= 1) & (nw >= 1)
    off[m] = (nh[m] - 1) * W + (nw[m] - 1)
    m = (nw == 0) & (nh >= 1)
    off[m] = _CORNER - nh[m]
    m = (nh == 0) & (nw == 0)
    off[m] = _CORNER
    m = (nh == 0) & (nw >= 1) & (nw <= 64)
    off[m] = _RUNA + nw[m] - 1
    m = (nh == 0) & (nw >= 65)
    off[m] = (_COLB + 128) - nw[m]
    colb = (np.arange(64, dtype=np.int32) * W + (W - 1))
    misc = np.empty((32,), np.int32)
    misc[0:16] = 127
    misc[0:4] = (127, 8064, _COLB, _LEFT)
    misc[16:32] = 63
    misc[16:18] = (63, 1984)
    bnd = [(0, j) for j in range(Ww)] + [(i, 0) for i in range(1, Hh)]
    bnd.append((Hh - 1, 0))
    bnd_off = np.stack([off[i, j] for (i, j) in bnd])
    bnd_pos = np.array([i * Ww + j for (i, j) in bnd], np.int32)
    return np.concatenate(
        [colb, misc, bnd_off.T.astype(np.int32).ravel(), bnd_pos]
    )


_IDX_TAB = _gather_table()
_COLB_OFF = 0
_MISC_OFF = 64
_BND_OFF = 96
_BNDPOS_OFF = 96 + 7 * 96
_TABLEN = _BNDPOS_OFF + 96
_DELTAS = (1, 128, 129, 130, 257, 258)


def _body(
    x_hbm, idx_hbm, out_hbm, idxv, buf0, buf1, cidx0, cidx1, ob0, ob1,
    sem0, sem1, osem0, osem1,
):
    wid = lax.axis_index("s") * 2 + lax.axis_index("c")
    pltpu.sync_copy(idx_hbm, idxv)
    zeros = jnp.zeros((16,), jnp.float32)
    sc_idx = idxv[pl.ds(_MISC_OFF, 16)]
    oc_idx = idxv[pl.ds(_MISC_OFF + 16, 16)]
    two_iota = lax.iota(jnp.int32, 16) * 2
    dvecs = [jnp.full((16,), d, jnp.int32) for d in _DELTAS]
    tbase = wid * TPW

    def neighbors(t):
        n = t // NCH
        ch = t - n * NCH
        r = ch // 5
        f = ch - r * 5
        fm1 = (f + 4) % 5
        base = n * NCH
        c_a = base + r * 5 + fm1
        c_b = base + ((r + 1) % R) * 5 + fm1
        c_l = base + ((r + 5) % R) * 5 + fm1
        return n, c_a, c_b, c_l

    def fetch_descs(t, buf, cidx, sem):
        n, c_a, c_b, c_l = neighbors(t)
        return [
            pltpu.make_async_copy(
                x_hbm.at[pl.ds(t * CHW, CHW)], buf.at[pl.ds(0, CHW)], sem
            ),
            pltpu.make_async_copy(
                x_hbm.at[pl.ds(c_a * CHW + (H - 1) * W + Ww, Ww)],
                buf.at[pl.ds(_RUNA, Ww)],
                sem,
            ),
            pltpu.make_async_copy(
                x_hbm.at[pl.ds(c_l * CHW + (H - 1) * W, Ww)],
                buf.at[pl.ds(_LEFT, Ww)],
                sem,
            ),
            pltpu.make_async_copy(
                x_hbm.at[pl.ds(n * IMW, 8)], buf.at[pl.ds(_CORNER, 8)], sem
            ),
            pltpu.make_async_copy(x_hbm.at[cidx], buf.at[pl.ds(_COLB, 64)], sem),
        ]

    def issue(t, buf, cidx, sem):
        _, _, c_b, _ = neighbors(t)
        for q in range(4):
            cidx[pl.ds(q * 16, 16)] = (
                idxv[pl.ds(_COLB_OFF + q * 16, 16)] + c_b * CHW
            )
        for d in fetch_descs(t, buf, cidx, sem):
            d.start()

    def out_desc(t, ob, osem):
        return pltpu.make_async_copy(ob, out_hbm.at[pl.ds(t * NOUT, NOUT)], osem)

    def compute(k, t, buf, ob, osem):
        plsc.store_scatter(buf, [sc_idx], zeros)

        @pl.when(k > 0)
        def _():
            out_desc(t - 2, ob, osem).wait()

        @plsc.parallel_loop(0, NOUT // 16, unroll=8)
        def vb(v):
            i = v >> 2
            q = v & 3
            base = lax.max((2 * i - 1) * W + 32 * q - 1, 0)
            b0 = base + two_iota
            g = [plsc.load_gather(buf, [b0])]
            for d in dvecs:
                g.append(plsc.load_gather(buf, [b0 + d]))
            acc = ((g[0] + g[1]) + (g[2] + g[3])) + ((g[4] + g[5]) + g[6])
            ob[pl.ds(v * 16, 16)] = acc * np.float32(1.0 / 7.0)

        for b in range(6):
            g = []
            for kk in range(7):
                fi = idxv[pl.ds(_BND_OFF + kk * 96 + b * 16, 16)]
                g.append(plsc.load_gather(buf, [fi]))
            acc = ((g[0] + g[1]) + (g[2] + g[3])) + ((g[4] + g[5]) + g[6])
            pos = idxv[pl.ds(_BNDPOS_OFF + b * 16, 16)]
            plsc.store_scatter(ob, [pos], acc * np.float32(1.0 / 7.0))
        plsc.store_scatter(ob, [oc_idx], zeros)
        out_desc(t, ob, osem).start()

    issue(tbase, buf0, cidx0, sem0)

    def pair(k, carry):
        t0 = tbase + 2 * k
        t1 = t0 + 1
        issue(t1, buf1, cidx1, sem1)
        for d in fetch_descs(t0, buf0, cidx0, sem0):
            d.wait()
        compute(k, t0, buf0, ob0, osem0)

        @pl.when(k < TPW // 2 - 1)
        def _():
            issue(t0 + 2, buf0, cidx0, sem0)

        for d in fetch_descs(t1, buf1, cidx1, sem1):
            d.wait()
        compute(k, t1, buf1, ob1, osem1)
        return carry

    lax.fori_loop(0, TPW // 2, pair, 0)
    out_desc(tbase + TPW - 2, ob0, osem0).wait()
    out_desc(tbase + TPW - 1, ob1, osem1).wait()


@jax.jit
def kernel(x):
    x1 = x.reshape(NTASK * CHW)
    idx = jnp.asarray(_IDX_TAB)
    run = pl.kernel(
        _body,
        out_type=jax.ShapeDtypeStruct((NTASK * NOUT,), jnp.float32),
        mesh=plsc.VectorSubcoreMesh(core_axis_name="c", subcore_axis_name="s"),
        compiler_params=pltpu.CompilerParams(
            use_tc_tiling_on_sc=False, needs_layout_passes=False
        ),
        scratch_types=[
            pltpu.VMEM((_TABLEN,), jnp.int32),
            pltpu.VMEM((BUF,), jnp.float32),
            pltpu.VMEM((BUF,), jnp.float32),
            pltpu.VMEM((64,), jnp.int32),
            pltpu.VMEM((64,), jnp.int32),
            pltpu.VMEM((NOUT,), jnp.float32),
            pltpu.VMEM((NOUT,), jnp.float32),
            pltpu.SemaphoreType.DMA,
            pltpu.SemaphoreType.DMA,
            pltpu.SemaphoreType.DMA,
            pltpu.SemaphoreType.DMA,
        ],
    )
    out = run(x1, idx)
    return out.reshape(8, 32, R, 5, Hh, Ww)
